# Initial kernel scaffold; baseline (speedup 1.0000x reference)
import jax
import jax.numpy as jnp
from jax.experimental import pallas as pl


def kernel(x, token_table, pos_table):
    raise NotImplementedError("write your pallas kernel here")


# speedup vs baseline: 6.3331x; 6.3331x over previous
import functools

import jax
import jax.numpy as jnp
from jax import lax
from jax.experimental import pallas as pl
from jax.experimental.pallas import tpu as pltpu
from jax.experimental.pallas import tpu_sc as plsc

B, L, D, V = 16384, 200, 32, 100000
NC, NS = 2, 16
NW = NC * NS
TOK = B * L
PER_W = TOK // NW
IDXW = 100
GPC = 16
CTOK = IDXW * GPC
NCH = PER_W // CTOK
NBUF = 2
LANES = 16

_mesh = plsc.VectorSubcoreMesh(
    core_axis_name="c", subcore_axis_name="s", num_cores=NC, num_subcores=NS
)


@functools.partial(
    pl.kernel,
    out_type=jax.ShapeDtypeStruct((TOK, D), jnp.float32),
    mesh=_mesh,
    compiler_params=pltpu.CompilerParams(use_tc_tiling_on_sc=False),
    scratch_types=[
        pltpu.VMEM((NBUF, GPC, IDXW), jnp.int32),
        pltpu.VMEM((NBUF, CTOK, D), jnp.float32),
        pltpu.VMEM((L * D,), jnp.float32),
        pltpu.SemaphoreType.DMA,
        pltpu.SemaphoreType.DMA,
        pltpu.SemaphoreType.DMA,
        pltpu.SemaphoreType.DMA,
    ],
)
def _emb_kernel(x2, tab, posf, out, idx_v, rows_v, pos_v, g0, g1, o0, o1):
    gsem = (g0, g1)
    osem = (o0, o1)
    wid = lax.axis_index("s") * NC + lax.axis_index("c")
    base_tok = wid * PER_W
    base_row = wid * (PER_W // IDXW)

    pltpu.sync_copy(posf, pos_v)

    def issue_gathers(c, b):
        pltpu.sync_copy(x2.at[pl.ds(base_row + c * GPC, GPC)], idx_v.at[b])
        for j in range(GPC):
            pltpu.make_async_copy(
                tab.at[idx_v.at[b].at[j]],
                rows_v.at[b].at[pl.ds(j * IDXW, IDXW)],
                gsem[b],
            ).start()

    def drain_gathers(b):
        pltpu.make_async_copy(tab.at[pl.ds(0, CTOK)], rows_v.at[b], gsem[b]).wait()

    def add_pos(b):
        rv = rows_v.at[b]

        def body_fn(r, carry):
            p0 = pos_v[pl.ds(r * D, LANES)]
            p1 = pos_v[pl.ds(r * D + LANES, LANES)]
            for rep in range(CTOK // L):
                row = rep * L + r
                rv[row, pl.ds(0, LANES)] = rv[row, pl.ds(0, LANES)] + p0
                rv[row, pl.ds(LANES, LANES)] = rv[row, pl.ds(LANES, LANES)] + p1
            return carry

        lax.fori_loop(0, L, body_fn, 0)

    def issue_store(c, b):
        pltpu.make_async_copy(
            rows_v.at[b], out.at[pl.ds(base_tok + c * CTOK, CTOK)], osem[b]
        ).start()

    def drain_store(c, b):
        pltpu.make_async_copy(
            rows_v.at[b], out.at[pl.ds(base_tok + c * CTOK, CTOK)], osem[b]
        ).wait()

    for b in range(NBUF):
        issue_gathers(b, b)

    def chunk_body(cc, carry):
        for b in range(NBUF):
            c = cc * NBUF + b
            drain_gathers(b)
            add_pos(b)
            issue_store(c, b)

            @pl.when(c + NBUF < NCH)
            def _prefetch():
                drain_store(c, b)
                issue_gathers(c + NBUF, b)

        return carry

    lax.fori_loop(0, NCH // NBUF, chunk_body, 0)

    for b in range(NBUF):
        drain_store(NCH - NBUF + b, b)


def kernel(x, token_table, pos_table):
    xf = x.reshape(-1).astype(jnp.int32).reshape(TOK // IDXW, IDXW)
    posf = pos_table.reshape(-1)
    out = _emb_kernel(xf, token_table, posf)
    return out.reshape(B, L, D)
